# Initial kernel scaffold; baseline (speedup 1.0000x reference)
import jax
import jax.numpy as jnp
from jax.experimental import pallas as pl


def kernel(x, edge_index, W1, b1, W2, b2):
    raise NotImplementedError("write your pallas kernel here")


# speedup vs baseline: 14.1371x; 14.1371x over previous
import functools

import jax
import jax.numpy as jnp
from jax import lax
from jax.experimental import pallas as pl
from jax.experimental.pallas import tpu as pltpu
from jax.experimental.pallas import tpu_sc as plsc

NC = 2
NS = 16
NW = NC * NS
CHUNK = 128
GROUP = 8


def _sc_scatter_fn(n_nodes, d, cpw, acc_rows):
  rows_per_tile = acc_rows // NS
  copies_per_tile = rows_per_tile // CHUNK
  n_groups = cpw // GROUP

  mesh = plsc.VectorSubcoreMesh(core_axis_name="c", subcore_axis_name="s")

  @functools.partial(
      pl.kernel,
      mesh=mesh,
      out_type=jax.ShapeDtypeStruct((NC, acc_rows, d), jnp.float32),
      scratch_types=[
          pltpu.VMEM((3, GROUP, 2, CHUNK), jnp.int32),
          pltpu.VMEM((2, CHUNK, d), jnp.float32),
          pltpu.VMEM_SHARED((acc_rows, d), jnp.float32),
          [pltpu.SemaphoreType.DMA] * 3,
          [pltpu.SemaphoreType.DMA] * 2,
          [pltpu.SemaphoreType.DMA] * 2,
      ],
  )
  def sc_kernel(idx_hbm, x_hbm, out_hbm, idx_v, msg, acc, sem_i, sem_g, sem_s):
    c = lax.axis_index("c")
    s = lax.axis_index("s")
    w = s * NC + c

    def idx_fetch(g):
      return pltpu.async_copy(
          idx_hbm.at[pl.ds(w * cpw + g * GROUP, GROUP)],
          idx_v.at[g % 3], sem_i[g % 3])

    def gather(j):
      return pltpu.async_copy(
          x_hbm.at[idx_v.at[(j // GROUP) % 3, j % GROUP, 1]],
          msg.at[j % 2], sem_g[j % 2])

    def scatter(j):
      return pltpu.async_copy(
          msg.at[j % 2],
          acc.at[idx_v.at[(j // GROUP) % 3, j % GROUP, 0]],
          sem_s[j % 2], add=True)

    d_g = [None] * cpw
    d_s = [None] * cpw
    d_i = [None] * n_groups
    d_i[0] = idx_fetch(0)

    zero = jnp.zeros((16,), jnp.float32)

    with jax.named_scope("zero_acc"):
      @pl.loop(0, CHUNK)
      def _(i):
        for k in range(d // 16):
          msg[0, i, pl.ds(k * 16, 16)] = zero

      base = s * rows_per_tile
      for k in range(copies_per_tile):
        pltpu.sync_copy(msg.at[0], acc.at[pl.ds(base + k * CHUNK, CHUNK)])

    with jax.named_scope("edge_loop"):
      d_i[0].wait()
      if n_groups > 1:
        d_i[1] = idx_fetch(1)
      d_g[0] = gather(0)
      if cpw > 1:
        d_g[1] = gather(1)
      plsc.subcore_barrier()

      for j in range(cpw):
        g, jj = divmod(j, GROUP)
        if g + 1 < n_groups and jj == 0 and g > 0:
          d_i[g + 1] = idx_fetch(g + 1)
        if j >= 1:
          d_s[j - 1].wait()
        if j + 1 < cpw:
          if (j + 1) % GROUP == 0:
            d_i[(j + 1) // GROUP].wait()
          if j + 1 > 1:
            d_g[j + 1] = gather(j + 1)
        d_g[j].wait()
        d_s[j] = scatter(j)

      d_s[cpw - 1].wait()
      plsc.subcore_barrier()

    with jax.named_scope("copy_out"):
      d_o = [None] * copies_per_tile
      for k in range(copies_per_tile):
        b = k % 2
        sl = pl.ds(base + k * CHUNK, CHUNK)
        if k >= 2:
          d_o[k - 2].wait()
        pltpu.async_copy(acc.at[sl], msg.at[b], sem_g[b]).wait()
        d_o[k] = pltpu.async_copy(msg.at[b], out_hbm.at[c, sl], sem_s[b])
      for k in range(max(copies_per_tile - 2, 0), copies_per_tile):
        d_o[k].wait()

  return sc_kernel


def _mlp_call(x, partials, w1, b1, w2, b2):
  n, d = x.shape
  dh = w1.shape[1]
  bm = 2000

  def body(x_ref, a0_ref, a1_ref, w1_ref, b1_ref, w2_ref, b2_ref, o_ref):
    out = x_ref[...] + a0_ref[0] + a1_ref[0]
    h = jnp.dot(out, w1_ref[...], preferred_element_type=jnp.float32)
    h = jnp.maximum(h + b1_ref[...], 0.0)
    o_ref[...] = (
        jnp.dot(h, w2_ref[...], preferred_element_type=jnp.float32)
        + b2_ref[...])

  return pl.pallas_call(
      body,
      grid=(n // bm,),
      in_specs=[
          pl.BlockSpec((bm, d), lambda i: (i, 0)),
          pl.BlockSpec((1, bm, d), lambda i: (0, i, 0)),
          pl.BlockSpec((1, bm, d), lambda i: (1, i, 0)),
          pl.BlockSpec((d, dh), lambda i: (0, 0)),
          pl.BlockSpec((1, dh), lambda i: (0, 0)),
          pl.BlockSpec((dh, d), lambda i: (0, 0)),
          pl.BlockSpec((1, d), lambda i: (0, 0)),
      ],
      out_specs=pl.BlockSpec((bm, d), lambda i: (i, 0)),
      out_shape=jax.ShapeDtypeStruct((n, d), jnp.float32),
  )(x, partials, partials, w1, b1, w2, b2)


def kernel(x, edge_index, W1, b1, W2, b2):
  n, d = x.shape
  e = edge_index.shape[1]

  rows_per_tile = -(-n // (NS * CHUNK)) * CHUNK
  acc_rows = rows_per_tile * NS
  nch = e // CHUNK
  cpw = -(-nch // (NW * GROUP)) * GROUP
  nch_pad = cpw * NW
  npad = nch_pad - nch

  ei3 = edge_index.astype(jnp.int32).reshape(2, nch, CHUNK).transpose(1, 0, 2)
  pad_rows = (n + jnp.arange(npad * CHUNK, dtype=jnp.int32)
              % (acc_rows - n)).reshape(npad, 1, CHUNK)
  pad_cols = (jnp.arange(npad * CHUNK, dtype=jnp.int32)
              % n).reshape(npad, 1, CHUNK)
  idx = jnp.concatenate(
      [ei3, jnp.concatenate([pad_rows, pad_cols], axis=1)], axis=0)

  partials = _sc_scatter_fn(n, d, cpw, acc_rows)(idx, x)

  return _mlp_call(x, partials, W1, b1.reshape(1, -1), W2, b2.reshape(1, -1))


# speedup vs baseline: 14.5371x; 1.0283x over previous
import functools

import jax
import jax.numpy as jnp
from jax import lax
from jax.experimental import pallas as pl
from jax.experimental.pallas import tpu as pltpu
from jax.experimental.pallas import tpu_sc as plsc

NC = 2
NS = 16
NW = NC * NS
CHUNK = 128
GROUP = 4


def _sc_scatter_fn(n_nodes, d, nch, acc_rows):
  rows_per_tile = acc_rows // NS
  copies_per_tile = rows_per_tile // CHUNK
  trash_rows = acc_rows - n_nodes

  total_groups = nch // GROUP
  base_groups = total_groups // NW
  extra = total_groups - base_groups * NW
  n_groups = base_groups + 1
  cpw = n_groups * GROUP

  mesh = plsc.VectorSubcoreMesh(core_axis_name="c", subcore_axis_name="s")

  @functools.partial(
      pl.kernel,
      mesh=mesh,
      out_type=jax.ShapeDtypeStruct((NC, acc_rows, d), jnp.float32),
      scratch_types=[
          pltpu.VMEM((3, GROUP, 2, CHUNK), jnp.int32),
          pltpu.VMEM((2, CHUNK, d), jnp.float32),
          pltpu.VMEM_SHARED((acc_rows, d), jnp.float32),
          [pltpu.SemaphoreType.DMA] * 3,
          [pltpu.SemaphoreType.DMA] * 2,
          [pltpu.SemaphoreType.DMA] * 2,
      ],
  )
  def sc_kernel(idx_hbm, x_hbm, out_hbm, idx_v, msg, acc, sem_i, sem_g, sem_s):
    c = lax.axis_index("c")
    s = lax.axis_index("s")
    w = s * NC + c
    start_chunk = GROUP * (base_groups * w + jnp.minimum(w, extra))

    def idx_fetch(g):
      off = jnp.minimum(start_chunk + g * GROUP, nch - GROUP)
      return pltpu.async_copy(
          idx_hbm.at[pl.ds(off, GROUP)], idx_v.at[g % 3], sem_i[g % 3])

    def gather(j):
      return pltpu.async_copy(
          x_hbm.at[idx_v.at[(j // GROUP) % 3, j % GROUP, 1]],
          msg.at[j % 2], sem_g[j % 2])

    def scatter(j):
      return pltpu.async_copy(
          msg.at[j % 2],
          acc.at[idx_v.at[(j // GROUP) % 3, j % GROUP, 0]],
          sem_s[j % 2], add=True)

    def trash_overwrite():
      @pl.when(w >= extra)
      def _():
        lane = lax.iota(jnp.int32, 16)
        base = w * (GROUP * CHUNK)
        ring = (n_groups - 1) % 3
        for jj in range(GROUP):
          for k in range(d // 16):
            vec = base + jj * CHUNK + k * 16 + lane
            idx_v[ring, jj, 0, pl.ds(k * 16, 16)] = (
                n_nodes + vec % trash_rows)

    d_g = [None] * cpw
    d_s = [None] * cpw
    d_i = [None] * n_groups
    d_i[0] = idx_fetch(0)

    zero = jnp.zeros((16,), jnp.float32)

    with jax.named_scope("zero_acc"):
      @pl.loop(0, CHUNK)
      def _(i):
        for k in range(d // 16):
          msg[0, i, pl.ds(k * 16, 16)] = zero

      base = s * rows_per_tile
      for k in range(copies_per_tile):
        pltpu.sync_copy(msg.at[0], acc.at[pl.ds(base + k * CHUNK, CHUNK)])

    with jax.named_scope("edge_loop"):
      d_i[0].wait()
      if n_groups > 1:
        d_i[1] = idx_fetch(1)
      d_g[0] = gather(0)
      if cpw > 1:
        d_g[1] = gather(1)
      plsc.subcore_barrier()

      for j in range(cpw):
        g, jj = divmod(j, GROUP)
        if g + 1 < n_groups and jj == 0 and g > 0:
          d_i[g + 1] = idx_fetch(g + 1)
        if j >= 1:
          d_s[j - 1].wait()
        if j + 1 < cpw:
          if (j + 1) % GROUP == 0:
            d_i[(j + 1) // GROUP].wait()
            if (j + 1) // GROUP == n_groups - 1:
              trash_overwrite()
          if j + 1 > 1:
            d_g[j + 1] = gather(j + 1)
        d_g[j].wait()
        d_s[j] = scatter(j)

      d_s[cpw - 1].wait()
      plsc.subcore_barrier()

    with jax.named_scope("copy_out"):
      d_o = [None] * copies_per_tile
      for k in range(copies_per_tile):
        b = k % 2
        sl = pl.ds(base + k * CHUNK, CHUNK)
        if k >= 2:
          d_o[k - 2].wait()
        pltpu.async_copy(acc.at[sl], msg.at[b], sem_g[b]).wait()
        d_o[k] = pltpu.async_copy(msg.at[b], out_hbm.at[c, sl], sem_s[b])
      for k in range(max(copies_per_tile - 2, 0), copies_per_tile):
        d_o[k].wait()

  return sc_kernel


def _mlp_call(x, partials, w1, b1, w2, b2):
  n, d = x.shape
  dh = w1.shape[1]
  bm = 2000

  def body(x_ref, a0_ref, a1_ref, w1_ref, b1_ref, w2_ref, b2_ref, o_ref):
    out = x_ref[...] + a0_ref[0] + a1_ref[0]
    h = jnp.dot(out, w1_ref[...], preferred_element_type=jnp.float32)
    h = jnp.maximum(h + b1_ref[...], 0.0)
    o_ref[...] = (
        jnp.dot(h, w2_ref[...], preferred_element_type=jnp.float32)
        + b2_ref[...])

  return pl.pallas_call(
      body,
      grid=(n // bm,),
      in_specs=[
          pl.BlockSpec((bm, d), lambda i: (i, 0)),
          pl.BlockSpec((1, bm, d), lambda i: (0, i, 0)),
          pl.BlockSpec((1, bm, d), lambda i: (1, i, 0)),
          pl.BlockSpec((d, dh), lambda i: (0, 0)),
          pl.BlockSpec((1, dh), lambda i: (0, 0)),
          pl.BlockSpec((dh, d), lambda i: (0, 0)),
          pl.BlockSpec((1, d), lambda i: (0, 0)),
      ],
      out_specs=pl.BlockSpec((bm, d), lambda i: (i, 0)),
      out_shape=jax.ShapeDtypeStruct((n, d), jnp.float32),
  )(x, partials, partials, w1, b1, w2, b2)


def kernel(x, edge_index, W1, b1, W2, b2):
  n, d = x.shape
  e = edge_index.shape[1]

  rows_per_tile = -(-n // (NS * CHUNK)) * CHUNK
  acc_rows = rows_per_tile * NS
  nch = e // CHUNK

  ei3 = edge_index.astype(jnp.int32).reshape(2, nch, CHUNK).transpose(1, 0, 2)

  partials = _sc_scatter_fn(n, d, nch, acc_rows)(ei3, x)

  return _mlp_call(x, partials, W1, b1.reshape(1, -1), W2, b2.reshape(1, -1))
